# baseline (speedup 1.0000x reference)
import functools

import numpy as np
import jax
import jax.numpy as jnp
from jax import lax
from jax.experimental import pallas as pl
from jax.experimental.pallas import tpu as pltpu
from jax.experimental.pallas import tpu_sc as plsc

N = 10000
E = 320000
D = 128
DH = D // 2

NC = 2
NS = 16
C = 128
CPT = 160
E_PAD = NS * CPT * C
N_PAD = 10112
STRIPE = N_PAD // NS
ROW_BLK = 1000

K = 5
G = CPT // K


def _mlp_in_body(x_ref, w0, b0, w1, b1, w2, b2, hid_ref, msg_ref):
    x = x_ref[...].astype(jnp.bfloat16)
    h = jnp.maximum(jnp.dot(x, w0[...], preferred_element_type=jnp.float32) + b0[...], 0.0)
    hid_ref[...] = h
    hb = h.astype(jnp.bfloat16)
    m1 = jnp.maximum(jnp.dot(hb, w1[...], preferred_element_type=jnp.float32) + b1[...], 0.0)
    msg = jnp.maximum(jnp.dot(m1.astype(jnp.bfloat16), w2[...],
                              preferred_element_type=jnp.float32) + b2[...], 0.0)
    msg_ref[0] = msg[:, :DH].astype(jnp.bfloat16)
    msg_ref[1] = msg[:, DH:].astype(jnp.bfloat16)


def _mlp_out_body(fl_ref, fr_ref, hid_ref, w3, b3, w4, b4, out_ref):
    f = jnp.concatenate([fl_ref[0], fr_ref[0]], axis=1)
    a1 = jnp.maximum(jnp.dot(f, w3[...], preferred_element_type=jnp.float32) + b3[...], 0.0)
    out_ref[...] = (
        jnp.maximum(jnp.dot(a1.astype(jnp.bfloat16), w4[...],
                            preferred_element_type=jnp.float32) + b4[...], 0.0)
        + hid_ref[...]
    )


_row_spec = pl.BlockSpec((ROW_BLK, D), lambda i: (i, 0))
_msg_spec = pl.BlockSpec((2, ROW_BLK, DH), lambda i: (0, i, 0))
_hl_spec = pl.BlockSpec((1, ROW_BLK, DH), lambda i: (0, i, 0))
_hr_spec = pl.BlockSpec((1, ROW_BLK, DH), lambda i: (1, i, 0))
_w_spec = pl.BlockSpec((D, D), lambda i: (0, 0))
_b_spec = pl.BlockSpec((1, D), lambda i: (0, 0))

_mlp_in_call = pl.pallas_call(
    _mlp_in_body,
    grid=(N // ROW_BLK,),
    in_specs=[_row_spec, _w_spec, _b_spec, _w_spec, _b_spec, _w_spec, _b_spec],
    out_specs=[_row_spec, _msg_spec],
    out_shape=[jax.ShapeDtypeStruct((N, D), jnp.float32),
               jax.ShapeDtypeStruct((2, N, DH), jnp.bfloat16)],
)

_mlp_out_call = pl.pallas_call(
    _mlp_out_body,
    grid=(N // ROW_BLK,),
    in_specs=[_hl_spec, _hr_spec, _row_spec, _w_spec, _b_spec, _w_spec, _b_spec],
    out_specs=_row_spec,
    out_shape=jax.ShapeDtypeStruct((N, D), jnp.float32),
)


def _sc_body(msg_hbm, src_hbm, dst_hbm, zeros_hbm, out_hbm,
             src_v, dst_v, rows_v, acc_sh, gsem, ssem):
    c = lax.axis_index("c")
    s = lax.axis_index("s")

    pltpu.sync_copy(zeros_hbm.at[pl.ds(s * STRIPE, STRIPE)],
                    acc_sh.at[pl.ds(s * STRIPE, STRIPE)])

    pltpu.sync_copy(src_hbm.at[pl.ds(s * CPT, CPT)], src_v)
    pltpu.sync_copy(dst_hbm.at[pl.ds(s * CPT, CPT)], dst_v)

    bias = c * N

    def _bias_row(i, carry):
        for j in range(C // 16):
            src_v[i, pl.ds(j * 16, 16)] = src_v[i, pl.ds(j * 16, 16)] + bias
        return carry

    lax.fori_loop(0, CPT, _bias_row, 0, unroll=4)

    plsc.subcore_barrier()

    for b in range(K):
        pltpu.async_copy(msg_hbm.at[src_v.at[b]], rows_v.at[0, b], gsem)

    def body(g, carry):
        cur = lax.rem(g, 2)
        nxt = 1 - cur
        base = g * K
        nbase = jnp.minimum(base + K, CPT - K)
        for b in range(K):
            pltpu.make_async_copy(msg_hbm.at[pl.ds(0, C)], rows_v.at[cur, b], gsem).wait()
        for b in range(K):
            pltpu.async_copy(msg_hbm.at[src_v.at[nbase + b]], rows_v.at[nxt, b], gsem)
        @pl.when(g > 0)
        def _():
            for b in range(K):
                pltpu.make_async_copy(msg_hbm.at[pl.ds(0, C)],
                                      acc_sh.at[pl.ds(0, C)], ssem).wait()
        for b in range(K):
            pltpu.async_copy(rows_v.at[cur, b], acc_sh.at[dst_v.at[base + b]],
                             ssem, add=True)
        return carry

    lax.fori_loop(0, G, body, 0)

    for b in range(K):
        pltpu.make_async_copy(msg_hbm.at[pl.ds(0, C)],
                              acc_sh.at[pl.ds(0, C)], ssem).wait()
        pltpu.make_async_copy(msg_hbm.at[pl.ds(0, C)], rows_v.at[0, b], gsem).wait()

    plsc.subcore_barrier()

    pltpu.sync_copy(acc_sh.at[pl.ds(s * STRIPE, STRIPE)],
                    out_hbm.at[c, pl.ds(s * STRIPE, STRIPE)])


_sc_call = pl.kernel(
    _sc_body,
    mesh=plsc.VectorSubcoreMesh(core_axis_name="c", subcore_axis_name="s"),
    out_type=jax.ShapeDtypeStruct((NC, N_PAD, DH), jnp.bfloat16),
    scratch_types=[
        pltpu.VMEM((CPT, C), jnp.int32),
        pltpu.VMEM((CPT, C), jnp.int32),
        pltpu.VMEM((2, K, C, DH), jnp.bfloat16),
        pltpu.VMEM_SHARED((N_PAD, DH), jnp.bfloat16),
        pltpu.SemaphoreType.DMA,
        pltpu.SemaphoreType.DMA,
    ],
    compiler_params=pltpu.CompilerParams(use_tc_tiling_on_sc=False),
)


def kernel(feature, edge_index, W0, b0, W1, b1, W2, b2, W3, b3, W4, b4):
    bf = jnp.bfloat16
    hid, msg = _mlp_in_call(feature, W0.astype(bf), b0.reshape(1, D),
                            W1.astype(bf), b1.reshape(1, D),
                            W2.astype(bf), b2.reshape(1, D))

    msg2 = msg.reshape(2 * N, DH)

    pad = E_PAD - E
    src = jnp.concatenate([edge_index[0], jnp.zeros((pad,), jnp.int32)]).reshape(-1, C)
    dst = jnp.concatenate([edge_index[1], jnp.full((pad,), N_PAD - 1, jnp.int32)]).reshape(-1, C)
    zeros = jnp.zeros((N_PAD, DH), jnp.bfloat16)

    halves = _sc_call(msg2, src, dst, zeros)

    out = _mlp_out_call(halves, halves, hid,
                        W3.astype(bf), b3.reshape(1, D), W4.astype(bf), b4.reshape(1, D))
    return out


# speedup vs baseline: 1.1560x; 1.1560x over previous
import functools

import numpy as np
import jax
import jax.numpy as jnp
from jax import lax
from jax.experimental import pallas as pl
from jax.experimental.pallas import tpu as pltpu
from jax.experimental.pallas import tpu_sc as plsc

N = 10000
E = 320000
D = 128
DH = D // 2

NC = 2
NS = 16
C = 128
CPT = 160
E_PAD = NS * CPT * C
N_PAD = 10112
STRIPE = N_PAD // NS
ROW_BLK = 1000

K = 5
G = CPT // K


def _mlp_in_body(x_ref, w0, b0, w1, b1, w2, b2, hid_ref, msg_ref):
    x = x_ref[...]
    h = jnp.maximum(jnp.dot(x, w0[...], preferred_element_type=jnp.float32) + b0[...], 0.0)
    hid_ref[...] = h
    m1 = jnp.maximum(jnp.dot(h, w1[...], preferred_element_type=jnp.float32) + b1[...], 0.0)
    msg = jnp.maximum(jnp.dot(m1, w2[...], preferred_element_type=jnp.float32) + b2[...], 0.0)
    msg_ref[0] = msg[:, :DH].astype(jnp.bfloat16)
    msg_ref[1] = msg[:, DH:].astype(jnp.bfloat16)


def _mlp_out_body(fl_ref, fr_ref, hid_ref, w3, b3, w4, b4, out_ref):
    f = jnp.concatenate([fl_ref[0], fr_ref[0]], axis=1).astype(jnp.float32)
    a1 = jnp.maximum(jnp.dot(f, w3[...], preferred_element_type=jnp.float32) + b3[...], 0.0)
    out_ref[...] = (
        jnp.maximum(jnp.dot(a1, w4[...], preferred_element_type=jnp.float32) + b4[...], 0.0)
        + hid_ref[...]
    )


_row_spec = pl.BlockSpec((ROW_BLK, D), lambda i: (i, 0))
_msg_spec = pl.BlockSpec((2, ROW_BLK, DH), lambda i: (0, i, 0))
_hl_spec = pl.BlockSpec((1, ROW_BLK, DH), lambda i: (0, i, 0))
_hr_spec = pl.BlockSpec((1, ROW_BLK, DH), lambda i: (1, i, 0))
_w_spec = pl.BlockSpec((D, D), lambda i: (0, 0))
_b_spec = pl.BlockSpec((1, D), lambda i: (0, 0))

_mlp_in_call = pl.pallas_call(
    _mlp_in_body,
    grid=(N // ROW_BLK,),
    in_specs=[_row_spec, _w_spec, _b_spec, _w_spec, _b_spec, _w_spec, _b_spec],
    out_specs=[_row_spec, _msg_spec],
    out_shape=[jax.ShapeDtypeStruct((N, D), jnp.float32),
               jax.ShapeDtypeStruct((2, N, DH), jnp.bfloat16)],
)

_mlp_out_call = pl.pallas_call(
    _mlp_out_body,
    grid=(N // ROW_BLK,),
    in_specs=[_hl_spec, _hr_spec, _row_spec, _w_spec, _b_spec, _w_spec, _b_spec],
    out_specs=_row_spec,
    out_shape=jax.ShapeDtypeStruct((N, D), jnp.float32),
)


def _sc_body(msg_hbm, src_hbm, dst_hbm, zeros_hbm, out_hbm,
             src_v, dst_v, rows_v, acc_sh, gsem, ssem):
    c = lax.axis_index("c")
    s = lax.axis_index("s")

    pltpu.sync_copy(zeros_hbm.at[pl.ds(s * STRIPE, STRIPE)],
                    acc_sh.at[pl.ds(s * STRIPE, STRIPE)])

    pltpu.sync_copy(src_hbm.at[pl.ds(s * CPT, CPT)], src_v)
    pltpu.sync_copy(dst_hbm.at[pl.ds(s * CPT, CPT)], dst_v)

    bias = c * N

    def _bias_row(i, carry):
        for j in range(C // 16):
            src_v[i, pl.ds(j * 16, 16)] = src_v[i, pl.ds(j * 16, 16)] + bias
        return carry

    lax.fori_loop(0, CPT, _bias_row, 0, unroll=4)

    plsc.subcore_barrier()

    for b in range(K):
        pltpu.async_copy(msg_hbm.at[src_v.at[b]], rows_v.at[0, b], gsem)

    def body(g, carry):
        cur = lax.rem(g, 2)
        nxt = 1 - cur
        base = g * K
        nbase = jnp.minimum(base + K, CPT - K)
        for b in range(K):
            pltpu.make_async_copy(msg_hbm.at[pl.ds(0, C)], rows_v.at[cur, b], gsem).wait()
        for b in range(K):
            pltpu.async_copy(msg_hbm.at[src_v.at[nbase + b]], rows_v.at[nxt, b], gsem)
        @pl.when(g > 0)
        def _():
            for b in range(K):
                pltpu.make_async_copy(msg_hbm.at[pl.ds(0, C)],
                                      acc_sh.at[pl.ds(0, C)], ssem).wait()
        for b in range(K):
            pltpu.async_copy(rows_v.at[cur, b], acc_sh.at[dst_v.at[base + b]],
                             ssem, add=True)
        return carry

    lax.fori_loop(0, G, body, 0)

    for b in range(K):
        pltpu.make_async_copy(msg_hbm.at[pl.ds(0, C)],
                              acc_sh.at[pl.ds(0, C)], ssem).wait()
        pltpu.make_async_copy(msg_hbm.at[pl.ds(0, C)], rows_v.at[0, b], gsem).wait()

    plsc.subcore_barrier()

    pltpu.sync_copy(acc_sh.at[pl.ds(s * STRIPE, STRIPE)],
                    out_hbm.at[c, pl.ds(s * STRIPE, STRIPE)])


_sc_call = pl.kernel(
    _sc_body,
    mesh=plsc.VectorSubcoreMesh(core_axis_name="c", subcore_axis_name="s"),
    out_type=jax.ShapeDtypeStruct((NC, N_PAD, DH), jnp.bfloat16),
    scratch_types=[
        pltpu.VMEM((CPT, C), jnp.int32),
        pltpu.VMEM((CPT, C), jnp.int32),
        pltpu.VMEM((2, K, C, DH), jnp.bfloat16),
        pltpu.VMEM_SHARED((N_PAD, DH), jnp.bfloat16),
        pltpu.SemaphoreType.DMA,
        pltpu.SemaphoreType.DMA,
    ],
    compiler_params=pltpu.CompilerParams(use_tc_tiling_on_sc=False),
)


def kernel(feature, edge_index, W0, b0, W1, b1, W2, b2, W3, b3, W4, b4):
    hid, msg = _mlp_in_call(feature, W0, b0.reshape(1, D), W1, b1.reshape(1, D),
                            W2, b2.reshape(1, D))

    msg2 = msg.reshape(2 * N, DH)

    pad = E_PAD - E
    src = jnp.concatenate([edge_index[0], jnp.zeros((pad,), jnp.int32)]).reshape(-1, C)
    dst = jnp.concatenate([edge_index[1], jnp.full((pad,), N_PAD - 1, jnp.int32)]).reshape(-1, C)
    zeros = jnp.zeros((N_PAD, DH), jnp.bfloat16)

    halves = _sc_call(msg2, src, dst, zeros)

    out = _mlp_out_call(halves, halves, hid,
                        W3, b3.reshape(1, D), W4, b4.reshape(1, D))
    return out


# speedup vs baseline: 1.2179x; 1.0536x over previous
import functools

import numpy as np
import jax
import jax.numpy as jnp
from jax import lax
from jax.experimental import pallas as pl
from jax.experimental.pallas import tpu as pltpu
from jax.experimental.pallas import tpu_sc as plsc

N = 10000
E = 320000
D = 128
DH = D // 2

NC = 2
NS = 16
C = 128
CPT = 160
E_PAD = NS * CPT * C
N_PAD = 10112
STRIPE = N_PAD // NS
ROW_BLK = 1000

K = 5
G = CPT // K


def _mlp_in_body(x_ref, w0, b0, w1, b1, w2, b2, hid_ref, msg_ref):
    x = x_ref[...]
    h = jnp.maximum(jnp.dot(x, w0[...], preferred_element_type=jnp.float32) + b0[...], 0.0)
    hid_ref[...] = h
    m1 = jnp.maximum(jnp.dot(h, w1[...], preferred_element_type=jnp.float32) + b1[...], 0.0)
    msg = jnp.maximum(jnp.dot(m1, w2[...], preferred_element_type=jnp.float32) + b2[...], 0.0)
    msg_ref[0] = msg[:, :DH].astype(jnp.bfloat16)
    msg_ref[1] = msg[:, DH:].astype(jnp.bfloat16)


def _mlp_out_body(fl_ref, fr_ref, hid_ref, w3, b3, w4, b4, out_ref):
    f = jnp.concatenate([fl_ref[0], fr_ref[0]], axis=1).astype(jnp.float32)
    a1 = jnp.maximum(jnp.dot(f, w3[...], preferred_element_type=jnp.float32) + b3[...], 0.0)
    out_ref[...] = (
        jnp.maximum(jnp.dot(a1, w4[...], preferred_element_type=jnp.float32) + b4[...], 0.0)
        + hid_ref[...]
    )


_row_spec = pl.BlockSpec((ROW_BLK, D), lambda i: (i, 0))
_msg_spec = pl.BlockSpec((2, ROW_BLK, DH), lambda i: (0, i, 0))
_hl_spec = pl.BlockSpec((1, ROW_BLK, DH), lambda i: (0, i, 0))
_hr_spec = pl.BlockSpec((1, ROW_BLK, DH), lambda i: (1, i, 0))
_w_spec = pl.BlockSpec((D, D), lambda i: (0, 0))
_b_spec = pl.BlockSpec((1, D), lambda i: (0, 0))

_mlp_in_call = pl.pallas_call(
    _mlp_in_body,
    grid=(N // ROW_BLK,),
    in_specs=[_row_spec, _w_spec, _b_spec, _w_spec, _b_spec, _w_spec, _b_spec],
    out_specs=[_row_spec, _msg_spec],
    out_shape=[jax.ShapeDtypeStruct((N, D), jnp.float32),
               jax.ShapeDtypeStruct((2, N, DH), jnp.bfloat16)],
)

_mlp_out_call = pl.pallas_call(
    _mlp_out_body,
    grid=(N // ROW_BLK,),
    in_specs=[_hl_spec, _hr_spec, _row_spec, _w_spec, _b_spec, _w_spec, _b_spec],
    out_specs=_row_spec,
    out_shape=jax.ShapeDtypeStruct((N, D), jnp.float32),
)


def _sc_body(msg_hbm, src_hbm, dst_hbm, zeros_hbm, out_hbm,
             src_v, dst_v, rows_v, acc_sh, gsem, ssem):
    c = lax.axis_index("c")
    s = lax.axis_index("s")

    pltpu.sync_copy(zeros_hbm.at[pl.ds(s * STRIPE, STRIPE)],
                    acc_sh.at[pl.ds(s * STRIPE, STRIPE)])

    pltpu.sync_copy(src_hbm.at[pl.ds(s * CPT, CPT)], src_v)
    pltpu.sync_copy(dst_hbm.at[pl.ds(s * CPT, CPT)], dst_v)

    bias = c * N

    def _bias_row(i, carry):
        for j in range(C // 16):
            src_v[i, pl.ds(j * 16, 16)] = src_v[i, pl.ds(j * 16, 16)] + bias
        return carry

    lax.fori_loop(0, CPT, _bias_row, 0, unroll=4)

    plsc.subcore_barrier()

    for b in range(K):
        pltpu.async_copy(msg_hbm.at[src_v.at[b]], rows_v.at[0, b], gsem)

    def body(g, carry):
        cur = lax.rem(g, 2)
        nxt = 1 - cur
        base = g * K
        nbase = jnp.minimum(base + K, CPT - K)
        for b in range(K):
            pltpu.make_async_copy(msg_hbm.at[pl.ds(0, C)], rows_v.at[cur, b], gsem).wait()
            pltpu.async_copy(msg_hbm.at[src_v.at[nbase + b]], rows_v.at[nxt, b], gsem)
        @pl.when(g > 0)
        def _():
            for b in range(K):
                pltpu.make_async_copy(msg_hbm.at[pl.ds(0, C)],
                                      acc_sh.at[pl.ds(0, C)], ssem).wait()
        for b in range(K):
            pltpu.async_copy(rows_v.at[cur, b], acc_sh.at[dst_v.at[base + b]],
                             ssem, add=True)
        return carry

    lax.fori_loop(0, G, body, 0)

    for b in range(K):
        pltpu.make_async_copy(msg_hbm.at[pl.ds(0, C)],
                              acc_sh.at[pl.ds(0, C)], ssem).wait()
        pltpu.make_async_copy(msg_hbm.at[pl.ds(0, C)], rows_v.at[0, b], gsem).wait()

    plsc.subcore_barrier()

    pltpu.sync_copy(acc_sh.at[pl.ds(s * STRIPE, STRIPE)],
                    out_hbm.at[c, pl.ds(s * STRIPE, STRIPE)])


_sc_call = pl.kernel(
    _sc_body,
    mesh=plsc.VectorSubcoreMesh(core_axis_name="c", subcore_axis_name="s"),
    out_type=jax.ShapeDtypeStruct((NC, N_PAD, DH), jnp.bfloat16),
    scratch_types=[
        pltpu.VMEM((CPT, C), jnp.int32),
        pltpu.VMEM((CPT, C), jnp.int32),
        pltpu.VMEM((2, K, C, DH), jnp.bfloat16),
        pltpu.VMEM_SHARED((N_PAD, DH), jnp.bfloat16),
        pltpu.SemaphoreType.DMA,
        pltpu.SemaphoreType.DMA,
    ],
    compiler_params=pltpu.CompilerParams(use_tc_tiling_on_sc=False),
)


def kernel(feature, edge_index, W0, b0, W1, b1, W2, b2, W3, b3, W4, b4):
    hid, msg = _mlp_in_call(feature, W0, b0.reshape(1, D), W1, b1.reshape(1, D),
                            W2, b2.reshape(1, D))

    msg2 = msg.reshape(2 * N, DH)

    pad = E_PAD - E
    src = jnp.concatenate([edge_index[0], jnp.zeros((pad,), jnp.int32)]).reshape(-1, C)
    dst = jnp.concatenate([edge_index[1], jnp.full((pad,), N_PAD - 1, jnp.int32)]).reshape(-1, C)
    zeros = jnp.zeros((N_PAD, DH), jnp.bfloat16)

    halves = _sc_call(msg2, src, dst, zeros)

    out = _mlp_out_call(halves, halves, hid,
                        W3, b3.reshape(1, D), W4, b4.reshape(1, D))
    return out
